# initial kernel scaffold (unmeasured)
import jax
import jax.numpy as jnp
from jax import lax
from jax.experimental import pallas as pl
from jax.experimental.pallas import tpu as pltpu

N_DEV = 8
B_PER = 2
SQ = 512
HQ_PER = 8
DH = 64
D_MODEL = 768
D_HEADS = HQ_PER * DH
WINDOW = 128
SCALE = 0.125


def kernel(x, Wq, K_ext, V_ext, Wo):
    def body(x_ref, wq_ref, k_hbm, v_hbm, wo_ref, out_ref,
             wq_full, wo_full, k_loc, v_loc,
             kv_sems, swq, rwq, swo, rwo):
        my = lax.axis_index("i")
        left = lax.rem(my + N_DEV - 1, N_DEV)
        right = lax.rem(my + 1, N_DEV)

        ck = pltpu.make_async_copy(
            k_hbm.at[pl.ds(my * B_PER, B_PER)], k_loc, kv_sems.at[0])
        cv = pltpu.make_async_copy(
            v_hbm.at[pl.ds(my * B_PER, B_PER)], v_loc, kv_sems.at[1])
        ck.start()
        cv.start()

        bsem = pltpu.get_barrier_semaphore()
        pl.semaphore_signal(bsem, inc=1, device_id=(left,),
                            device_id_type=pl.DeviceIdType.MESH)
        pl.semaphore_signal(bsem, inc=1, device_id=(right,),
                            device_id_type=pl.DeviceIdType.MESH)
        pl.semaphore_wait(bsem, 2)

        wq_full[pl.ds(my, 1)] = wq_ref[...][None]
        wo_full[pl.ds(my, 1)] = wo_ref[...][None]

        ck.wait()
        cv.wait()

        x2d = x_ref[...].reshape(B_PER * SQ, D_MODEL)

        qi = lax.broadcasted_iota(jnp.int32, (SQ, SQ), 0)
        ki = lax.broadcasted_iota(jnp.int32, (SQ, SQ), 1)
        mask = jnp.abs(qi - ki) <= WINDOW

        accs = [jnp.zeros((SQ, D_MODEL), jnp.float32) for _ in range(B_PER)]

        for h in range(N_DEV):
            o = lax.rem(my + N_DEV - h, N_DEV)

            if h > 0:
                pltpu.make_async_remote_copy(
                    src_ref=wq_full.at[o], dst_ref=wq_full.at[o],
                    send_sem=swq.at[h - 1], recv_sem=rwq.at[h - 1],
                    device_id=(left,), device_id_type=pl.DeviceIdType.MESH,
                ).wait_recv()
                pltpu.make_async_remote_copy(
                    src_ref=wo_full.at[o], dst_ref=wo_full.at[o],
                    send_sem=swo.at[h - 1], recv_sem=rwo.at[h - 1],
                    device_id=(left,), device_id_type=pl.DeviceIdType.MESH,
                ).wait_recv()

            if h < N_DEV - 1:
                swq_d = pltpu.make_async_remote_copy(
                    src_ref=wq_full.at[o], dst_ref=wq_full.at[o],
                    send_sem=swq.at[h], recv_sem=rwq.at[h],
                    device_id=(right,), device_id_type=pl.DeviceIdType.MESH,
                )
                swq_d.start()
                swo_d = pltpu.make_async_remote_copy(
                    src_ref=wo_full.at[o], dst_ref=wo_full.at[o],
                    send_sem=swo.at[h], recv_sem=rwo.at[h],
                    device_id=(right,), device_id_type=pl.DeviceIdType.MESH,
                )
                swo_d.start()

            wq_blk = wq_full[pl.ds(o, 1)][0]
            wo_blk = wo_full[pl.ds(o, 1)][0]
            q_all = lax.dot_general(
                x2d, wq_blk, (((1,), (0,)), ((), ())),
                preferred_element_type=jnp.float32)

            for b in range(B_PER):
                qb = q_all[b * SQ:(b + 1) * SQ].reshape(SQ, HQ_PER, DH)
                kb = k_loc[b, :, pl.ds(o * HQ_PER, HQ_PER), :]
                vb = v_loc[b, :, pl.ds(o * HQ_PER, HQ_PER), :]
                ctx_heads = []
                for hh in range(HQ_PER):
                    qh = qb[:, hh, :]
                    kh = kb[:, hh, :]
                    vh = vb[:, hh, :]
                    s = lax.dot_general(
                        qh, kh, (((1,), (1,)), ((), ())),
                        preferred_element_type=jnp.float32) * SCALE
                    s = jnp.where(mask, s, -1e9)
                    s = s - jnp.max(s, axis=-1, keepdims=True)
                    w = jnp.exp(s)
                    w = w / jnp.sum(w, axis=-1, keepdims=True)
                    ctx_heads.append(lax.dot_general(
                        w, vh, (((1,), (0,)), ((), ())),
                        preferred_element_type=jnp.float32))
                ctx = jnp.concatenate(ctx_heads, axis=1)
                accs[b] = accs[b] + lax.dot_general(
                    ctx, wo_blk, (((1,), (0,)), ((), ())),
                    preferred_element_type=jnp.float32)

            if h < N_DEV - 1:
                swq_d.wait_send()
                swo_d.wait_send()

        for b in range(B_PER):
            out_ref[b] = accs[b]

    return pl.pallas_call(
        body,
        out_shape=jax.ShapeDtypeStruct((B_PER, SQ, D_MODEL), jnp.float32),
        in_specs=[
            pl.BlockSpec(memory_space=pltpu.VMEM),
            pl.BlockSpec(memory_space=pltpu.VMEM),
            pl.BlockSpec(memory_space=pltpu.ANY),
            pl.BlockSpec(memory_space=pltpu.ANY),
            pl.BlockSpec(memory_space=pltpu.VMEM),
        ],
        out_specs=pl.BlockSpec(memory_space=pltpu.VMEM),
        scratch_shapes=[
            pltpu.VMEM((N_DEV, D_MODEL, D_HEADS), jnp.float32),
            pltpu.VMEM((N_DEV, D_HEADS, D_MODEL), jnp.float32),
            pltpu.VMEM((B_PER, SQ, 64, DH), jnp.float32),
            pltpu.VMEM((B_PER, SQ, 64, DH), jnp.float32),
            pltpu.SemaphoreType.DMA((2,)),
            pltpu.SemaphoreType.DMA((N_DEV - 1,)),
            pltpu.SemaphoreType.DMA((N_DEV - 1,)),
            pltpu.SemaphoreType.DMA((N_DEV - 1,)),
            pltpu.SemaphoreType.DMA((N_DEV - 1,)),
        ],
        compiler_params=pltpu.CompilerParams(collective_id=0),
    )(x, Wq, K_ext, V_ext, Wo)


# baseline (device time: 741487 ns/iter reference)
import jax
import jax.numpy as jnp
from jax import lax
from jax.experimental import pallas as pl
from jax.experimental.pallas import tpu as pltpu

N_DEV = 8
B_PER = 2
SQ = 512
HQ_PER = 8
DH = 64
D_MODEL = 768
D_HEADS = HQ_PER * DH
WINDOW = 128
SCALE = 0.125


def kernel(x, Wq, K_ext, V_ext, Wo):
    def body(x_ref, wq_ref, k_hbm, v_hbm, wo_ref, out_ref,
             wq_full, wo_full, k_blk, v_blk, q_scr, ctx_scr,
             k_sems, v_sems, swq, rwq, swo, rwo):
        my = lax.axis_index("i")
        left = lax.rem(my + N_DEV - 1, N_DEV)
        right = lax.rem(my + 1, N_DEV)

        def start_kv_load(h):
            o_h = lax.rem(my + N_DEV - h, N_DEV)
            slot = h % 2
            for hh in range(HQ_PER):
                pltpu.make_async_copy(
                    k_hbm.at[pl.ds(my * B_PER, B_PER), :,
                             o_h * HQ_PER + hh, :],
                    k_blk.at[slot, hh], k_sems.at[slot]).start()
                pltpu.make_async_copy(
                    v_hbm.at[pl.ds(my * B_PER, B_PER), :,
                             o_h * HQ_PER + hh, :],
                    v_blk.at[slot, hh], v_sems.at[slot]).start()

        def wait_kv_load(h):
            slot = h % 2
            for hh in range(HQ_PER):
                pltpu.make_async_copy(
                    k_hbm.at[pl.ds(0, B_PER), :, 0, :],
                    k_blk.at[slot, hh], k_sems.at[slot]).wait()
                pltpu.make_async_copy(
                    v_hbm.at[pl.ds(0, B_PER), :, 0, :],
                    v_blk.at[slot, hh], v_sems.at[slot]).wait()

        start_kv_load(0)

        bsem = pltpu.get_barrier_semaphore()
        pl.semaphore_signal(bsem, inc=1, device_id=(left,),
                            device_id_type=pl.DeviceIdType.MESH)
        pl.semaphore_signal(bsem, inc=1, device_id=(right,),
                            device_id_type=pl.DeviceIdType.MESH)
        pl.semaphore_wait(bsem, 2)

        wq_full[pl.ds(my, 1)] = wq_ref[...][None]
        wo_full[pl.ds(my, 1)] = wo_ref[...][None]

        x2d = x_ref[...].reshape(B_PER * SQ, D_MODEL)

        qi = lax.broadcasted_iota(jnp.int32, (SQ, SQ), 0)
        ki = lax.broadcasted_iota(jnp.int32, (SQ, SQ), 1)
        mask = jnp.abs(qi - ki) <= WINDOW

        for h in range(N_DEV):
            o = lax.rem(my + N_DEV - h, N_DEV)
            slot = h % 2

            if h + 1 < N_DEV:
                start_kv_load(h + 1)

            if h > 0:
                pltpu.make_async_remote_copy(
                    src_ref=wq_full.at[o], dst_ref=wq_full.at[o],
                    send_sem=swq.at[h - 1], recv_sem=rwq.at[h - 1],
                    device_id=(left,), device_id_type=pl.DeviceIdType.MESH,
                ).wait_recv()
                pltpu.make_async_remote_copy(
                    src_ref=wo_full.at[o], dst_ref=wo_full.at[o],
                    send_sem=swo.at[h - 1], recv_sem=rwo.at[h - 1],
                    device_id=(left,), device_id_type=pl.DeviceIdType.MESH,
                ).wait_recv()

            if h < N_DEV - 1:
                swq_d = pltpu.make_async_remote_copy(
                    src_ref=wq_full.at[o], dst_ref=wq_full.at[o],
                    send_sem=swq.at[h], recv_sem=rwq.at[h],
                    device_id=(right,), device_id_type=pl.DeviceIdType.MESH,
                )
                swq_d.start()
                swo_d = pltpu.make_async_remote_copy(
                    src_ref=wo_full.at[o], dst_ref=wo_full.at[o],
                    send_sem=swo.at[h], recv_sem=rwo.at[h],
                    device_id=(right,), device_id_type=pl.DeviceIdType.MESH,
                )
                swo_d.start()

            wq_blk = wq_full[pl.ds(o, 1)][0]
            wo_blk = wo_full[pl.ds(o, 1)][0]
            q_scr[...] = lax.dot_general(
                x2d, wq_blk, (((1,), (0,)), ((), ())),
                preferred_element_type=jnp.float32)

            wait_kv_load(h)

            n_pair = HQ_PER // 2

            def pair_body(t, carry):
                b = t // n_pair
                pair = lax.rem(t, n_pair)
                q2 = q_scr[pl.ds(b * SQ, SQ), pl.ds(pair * 2 * DH, 2 * DH)]
                ctx_subs = []
                for sub in range(2):
                    hh = pair * 2 + sub
                    qh = q2[:, sub * DH:(sub + 1) * DH]
                    kh = k_blk[slot, pl.ds(hh, 1),
                               pl.ds(b, 1), :, :].reshape(SQ, DH)
                    vh = v_blk[slot, pl.ds(hh, 1),
                               pl.ds(b, 1), :, :].reshape(SQ, DH)
                    s = lax.dot_general(
                        qh, kh, (((1,), (1,)), ((), ())),
                        preferred_element_type=jnp.float32) * SCALE
                    s = jnp.where(mask, s, -1e9)
                    s = s - jnp.max(s, axis=-1, keepdims=True)
                    w = jnp.exp(s)
                    w = w / jnp.sum(w, axis=-1, keepdims=True)
                    ctx_subs.append(lax.dot_general(
                        w, vh, (((1,), (0,)), ((), ())),
                        preferred_element_type=jnp.float32))
                ctx_pair = jnp.concatenate(ctx_subs, axis=1)
                ctx_scr[pl.ds(b, 1), :,
                        pl.ds(pair * 2 * DH, 2 * DH)] = ctx_pair[None]
                return carry

            lax.fori_loop(0, B_PER * n_pair, pair_body, 0)

            for b in range(B_PER):
                part = lax.dot_general(
                    ctx_scr[b], wo_blk, (((1,), (0,)), ((), ())),
                    preferred_element_type=jnp.float32)
                if h == 0:
                    out_ref[b] = part
                else:
                    out_ref[b] = out_ref[b] + part

            if h < N_DEV - 1:
                swq_d.wait_send()
                swo_d.wait_send()

    return pl.pallas_call(
        body,
        out_shape=jax.ShapeDtypeStruct((B_PER, SQ, D_MODEL), jnp.float32),
        in_specs=[
            pl.BlockSpec(memory_space=pltpu.VMEM),
            pl.BlockSpec(memory_space=pltpu.VMEM),
            pl.BlockSpec(memory_space=pl.ANY),
            pl.BlockSpec(memory_space=pl.ANY),
            pl.BlockSpec(memory_space=pltpu.VMEM),
        ],
        out_specs=pl.BlockSpec(memory_space=pltpu.VMEM),
        scratch_shapes=[
            pltpu.VMEM((N_DEV, D_MODEL, D_HEADS), jnp.float32),
            pltpu.VMEM((N_DEV, D_HEADS, D_MODEL), jnp.float32),
            pltpu.VMEM((2, HQ_PER, B_PER, SQ, DH), jnp.float32),
            pltpu.VMEM((2, HQ_PER, B_PER, SQ, DH), jnp.float32),
            pltpu.VMEM((B_PER * SQ, D_HEADS), jnp.float32),
            pltpu.VMEM((B_PER, SQ, D_HEADS), jnp.float32),
            pltpu.SemaphoreType.DMA((2,)),
            pltpu.SemaphoreType.DMA((2,)),
            pltpu.SemaphoreType.DMA((N_DEV - 1,)),
            pltpu.SemaphoreType.DMA((N_DEV - 1,)),
            pltpu.SemaphoreType.DMA((N_DEV - 1,)),
            pltpu.SemaphoreType.DMA((N_DEV - 1,)),
        ],
        compiler_params=pltpu.CompilerParams(
            collective_id=0, vmem_limit_bytes=100 * 1024 * 1024),
    )(x, Wq, K_ext, V_ext, Wo)
